# baseline (speedup 1.0000x reference)
import functools

import jax
import jax.numpy as jnp
import numpy as np
from jax import lax
from jax.experimental import pallas as pl
from jax.experimental.pallas import tpu as pltpu
from jax.experimental.pallas import tpu_sc as plsc

VOCAB = 20000
EMBED = 50
HIDDEN = 100
BATCH = 4096
SEQLEN = 200
BN_EPS = 1e-5

NC = 2
NS = 16
LANES = 16
NW = NC * NS
BPW = BATCH // NW
CH0 = 104
CH1 = SEQLEN - CH0
NBUF = 4
EPAD = 64
TW = EPAD // 2
HPAD = 128

_PERM = np.concatenate([
    np.arange(0, 32, 2), np.arange(1, 32, 2),
    np.arange(32, 64, 2), np.arange(33, 64, 2),
])


def _sc_pool(table_i32, xt):
    mesh = plsc.VectorSubcoreMesh(core_axis_name="c", subcore_axis_name="s")

    @functools.partial(
        pl.kernel,
        out_type=jax.ShapeDtypeStruct((BATCH // 2, 2 * EPAD), jnp.float32),
        mesh=mesh,
        scratch_types=[
            pltpu.VMEM((SEQLEN, BPW), jnp.int32),
            pltpu.VMEM((BPW, SEQLEN), jnp.int32),
            *[pltpu.VMEM((SEQLEN, TW), jnp.int32) for _ in range(NBUF)],
            pltpu.VMEM((BPW // 2, 2 * EPAD), jnp.float32),
            *[pltpu.SemaphoreType.DMA for _ in range(NBUF)],
        ],
        compiler_params=pltpu.CompilerParams(
            needs_layout_passes=False, use_tc_tiling_on_sc=False
        ),
    )
    def pool(table_hbm, xt_hbm, out_hbm, raw_v, idx_v, *rest):
        bufs = rest[:NBUF]
        acc_v = rest[NBUF]
        sems = rest[NBUF + 1:]
        wid = lax.axis_index("s") * NC + lax.axis_index("c")
        base = wid * BPW

        pltpu.sync_copy(xt_hbm.at[:, pl.ds(base, BPW)], raw_v)

        iot = lax.iota(jnp.int32, LANES)

        def tr_body(r, _):
            for l0 in list(range(0, SEQLEN - LANES, LANES)) + [SEQLEN - LANES]:
                idx_v[r, pl.ds(l0, LANES)] = plsc.load_gather(
                    raw_v, [l0 + iot, jnp.broadcast_to(r, (LANES,))]
                )
            return 0

        lax.fori_loop(0, BPW, tr_body, 0)

        def fire(e, b):
            pltpu.async_copy(
                table_hbm.at[idx_v.at[e, pl.ds(0, CH0)]],
                bufs[b].at[pl.ds(0, CH0)],
                sems[b],
            )
            pltpu.async_copy(
                table_hbm.at[idx_v.at[e, pl.ds(CH0, CH1)]],
                bufs[b].at[pl.ds(CH0, CH1)],
                sems[b],
            )

        for b in range(NBUF):
            fire(b, b)

        inv = jnp.float32(1.0 / SEQLEN)
        zero = jnp.zeros((LANES,), jnp.float32)

        def reduce_block(buf):
            def grp(g, carry):
                a0, a1, a2, a3 = carry
                r0 = g * 8
                for r in range(8):
                    c0 = plsc.bitcast(buf[r0 + r, pl.ds(0, LANES)], jnp.bfloat16)
                    c1 = plsc.bitcast(buf[r0 + r, pl.ds(LANES, LANES)], jnp.bfloat16)
                    e0, o0 = plsc.unpack(c0, format=plsc.PackFormat.INTERLEAVED)
                    e1, o1 = plsc.unpack(c1, format=plsc.PackFormat.INTERLEAVED)
                    a0 = a0 + e0
                    a1 = a1 + o0
                    a2 = a2 + e1
                    a3 = a3 + o1
                return a0, a1, a2, a3

            return lax.fori_loop(0, SEQLEN // 8, grp, (zero, zero, zero, zero))

        def outer(i, _):
            for b in range(NBUF):
                e = i * NBUF + b
                pltpu.make_async_copy(
                    table_hbm.at[pl.ds(0, SEQLEN)], bufs[b], sems[b]
                ).wait()
                a0, a1, a2, a3 = reduce_block(bufs[b])

                @pl.when(e + NBUF < BPW)
                def _():
                    fire(e + NBUF, b)

                row = i * (NBUF // 2) + (b // 2)
                col0 = (b % 2) * EPAD
                acc_v[row, pl.ds(col0, LANES)] = a0 * inv
                acc_v[row, pl.ds(col0 + LANES, LANES)] = a1 * inv
                acc_v[row, pl.ds(col0 + 2 * LANES, LANES)] = a2 * inv
                acc_v[row, pl.ds(col0 + 3 * LANES, LANES)] = a3 * inv
            return 0

        lax.fori_loop(0, BPW // NBUF, outer, 0)
        pltpu.sync_copy(acc_v, out_hbm.at[pl.ds(wid * (BPW // 2), BPW // 2)])

    return pool(table_i32, xt)


def _mlp(pooled2, w1p, aux):
    def body(p_ref, w1_ref, aux_ref, o_ref):
        h = p_ref[...]
        h = jnp.where(h >= 0, h, 0.01 * h)
        b1 = aux_ref[0:1, :]
        s = aux_ref[1:2, :] * lax.rsqrt(aux_ref[4:5, :] + BN_EPS)
        t = aux_ref[2:3, :] - aux_ref[3:4, :] * s
        w2 = aux_ref[5:6, :]
        b2v = aux_ref[6:7, 0:1]

        def head(hh):
            h1 = jnp.dot(hh, w1_ref[...], preferred_element_type=jnp.float32) + b1
            h1 = jnp.where(h1 >= 0, h1, 0.01 * h1)
            h1 = h1 * s + t
            return jnp.sum(h1 * w2, axis=1, keepdims=True) + b2v

        o_ref[:, 0:1] = head(h[:, 0:EPAD])
        o_ref[:, 1:2] = head(h[:, EPAD:2 * EPAD])

    grid = 8
    bb = BATCH // 2 // grid
    return pl.pallas_call(
        body,
        grid=(grid,),
        in_specs=[
            pl.BlockSpec((bb, 2 * EPAD), lambda i: (i, 0)),
            pl.BlockSpec((EPAD, HPAD), lambda i: (0, 0)),
            pl.BlockSpec((8, HPAD), lambda i: (0, 0)),
        ],
        out_specs=pl.BlockSpec((bb, 2), lambda i: (i, 0)),
        out_shape=jax.ShapeDtypeStruct((BATCH // 2, 2), jnp.float32),
    )(pooled2, w1p, aux)


def kernel(x, embed_table, W1, b1, bn_scale, bn_bias, bn_mean, bn_var, W2, b2):
    f32 = jnp.float32
    xt = x.astype(jnp.int32).T

    t16 = jnp.pad(embed_table.astype(jnp.bfloat16), ((0, 0), (0, EPAD - EMBED)))
    ti32 = lax.bitcast_convert_type(t16.reshape(VOCAB, TW, 2), jnp.int32)
    pooled2 = _sc_pool(ti32, xt)

    w1p = jnp.zeros((EPAD, HPAD), f32).at[:EMBED, :HIDDEN].set(W1)
    w1p = w1p[jnp.asarray(_PERM), :]

    pad1 = (0, HPAD - HIDDEN)
    aux = jnp.stack([
        jnp.pad(b1, pad1),
        jnp.pad(bn_scale, pad1),
        jnp.pad(bn_bias, pad1),
        jnp.pad(bn_mean, pad1),
        jnp.pad(bn_var, pad1, constant_values=1.0),
        jnp.pad(W2[:, 0], pad1),
        jnp.full((HPAD,), b2[0], dtype=f32),
        jnp.zeros((HPAD,), dtype=f32),
    ])

    out = _mlp(pooled2, w1p, aux)
    return out.reshape(BATCH)


# speedup vs baseline: 1.1413x; 1.1413x over previous
import functools

import jax
import jax.numpy as jnp
import numpy as np
from jax import lax
from jax.experimental import pallas as pl
from jax.experimental.pallas import tpu as pltpu
from jax.experimental.pallas import tpu_sc as plsc

VOCAB = 20000
EMBED = 50
HIDDEN = 100
BATCH = 4096
SEQLEN = 200
BN_EPS = 1e-5

NC = 2
NS = 16
LANES = 16
NW = NC * NS
BPW = BATCH // NW
NBUF = 4
EPAD = 64
HPAD = 128

CHUNK_STARTS = list(range(0, SEQLEN - LANES, LANES)) + [SEQLEN - LANES]
NCHUNK = len(CHUNK_STARTS)
BUF_ROWS = NCHUNK * LANES

_PERM = np.concatenate([
    np.arange(0, 32, 2), np.arange(1, 32, 2),
    np.arange(32, 64, 2), np.arange(33, 64, 2),
])


def _sc_pool(table, xt):
    mesh = plsc.VectorSubcoreMesh(core_axis_name="c", subcore_axis_name="s")

    @functools.partial(
        pl.kernel,
        out_type=jax.ShapeDtypeStruct((BATCH // 2, 2 * EPAD), jnp.float32),
        mesh=mesh,
        scratch_types=[
            pltpu.VMEM((SEQLEN, BPW), jnp.int32),
            *[pltpu.VMEM((BUF_ROWS, EPAD), jnp.bfloat16) for _ in range(NBUF)],
            pltpu.VMEM((BPW // 2, 2 * EPAD), jnp.float32),
            *[pltpu.SemaphoreType.DMA for _ in range(NBUF)],
        ],
        compiler_params=pltpu.CompilerParams(
            needs_layout_passes=False, use_tc_tiling_on_sc=False
        ),
    )
    def pool(table_hbm, xt_hbm, out_hbm, raw_v, *rest):
        bufs = rest[:NBUF]
        acc_v = rest[NBUF]
        sems = rest[NBUF + 1:]
        wid = lax.axis_index("s") * NC + lax.axis_index("c")
        base = wid * BPW

        pltpu.sync_copy(xt_hbm.at[:, pl.ds(base, BPW)], raw_v)

        iot = lax.iota(jnp.int32, LANES)

        def fire(e, b):
            for ci, l0 in enumerate(CHUNK_STARTS):
                idx = plsc.load_gather(
                    raw_v, [l0 + iot, jnp.broadcast_to(e, (LANES,))]
                )
                pltpu.async_copy(
                    table_hbm.at[idx],
                    bufs[b].at[pl.ds(ci * LANES, LANES)],
                    sems[b],
                )

        for b in range(NBUF):
            fire(jnp.int32(b), b)

        inv = jnp.float32(1.0 / SEQLEN)
        zero = jnp.zeros((LANES,), jnp.float32)

        def reduce_block(buf):
            def grp(g, carry):
                a0, a1, a2, a3 = carry
                r0 = g * 8
                for r in range(8):
                    c0 = buf[r0 + r, pl.ds(0, 2 * LANES)]
                    c1 = buf[r0 + r, pl.ds(2 * LANES, 2 * LANES)]
                    e0, o0 = plsc.unpack(c0, format=plsc.PackFormat.INTERLEAVED)
                    e1, o1 = plsc.unpack(c1, format=plsc.PackFormat.INTERLEAVED)
                    a0 = a0 + e0
                    a1 = a1 + o0
                    a2 = a2 + e1
                    a3 = a3 + o1
                return a0, a1, a2, a3

            s_main = lax.fori_loop(0, 192 // 8, grp, (zero, zero, zero, zero))
            return lax.fori_loop(200 // 8, BUF_ROWS // 8, grp, s_main)

        def outer(i, _):
            for b in range(NBUF):
                e = i * NBUF + b
                pltpu.make_async_copy(
                    table_hbm.at[pl.ds(0, BUF_ROWS)], bufs[b], sems[b]
                ).wait()
                a0, a1, a2, a3 = reduce_block(bufs[b])

                @pl.when(e + NBUF < BPW)
                def _():
                    fire(e + NBUF, b)

                row = i * (NBUF // 2) + (b // 2)
                col0 = (b % 2) * EPAD
                acc_v[row, pl.ds(col0, LANES)] = a0 * inv
                acc_v[row, pl.ds(col0 + LANES, LANES)] = a1 * inv
                acc_v[row, pl.ds(col0 + 2 * LANES, LANES)] = a2 * inv
                acc_v[row, pl.ds(col0 + 3 * LANES, LANES)] = a3 * inv
            return 0

        lax.fori_loop(0, BPW // NBUF, outer, 0)
        pltpu.sync_copy(acc_v, out_hbm.at[pl.ds(wid * (BPW // 2), BPW // 2)])

    return pool(table, xt)


def _mlp(pooled2, w1p, aux):
    def body(p_ref, w1_ref, aux_ref, o_ref):
        h = p_ref[...]
        h = jnp.where(h >= 0, h, 0.01 * h)
        b1 = aux_ref[0:1, :]
        s = aux_ref[1:2, :] * lax.rsqrt(aux_ref[4:5, :] + BN_EPS)
        t = aux_ref[2:3, :] - aux_ref[3:4, :] * s
        w2 = aux_ref[5:6, :]
        b2v = aux_ref[6:7, 0:1]

        def head(hh):
            h1 = jnp.dot(hh, w1_ref[...], preferred_element_type=jnp.float32) + b1
            h1 = jnp.where(h1 >= 0, h1, 0.01 * h1)
            h1 = h1 * s + t
            return jnp.sum(h1 * w2, axis=1, keepdims=True) + b2v

        o_ref[:, 0:1] = head(h[:, 0:EPAD])
        o_ref[:, 1:2] = head(h[:, EPAD:2 * EPAD])

    grid = 8
    bb = BATCH // 2 // grid
    return pl.pallas_call(
        body,
        grid=(grid,),
        in_specs=[
            pl.BlockSpec((bb, 2 * EPAD), lambda i: (i, 0)),
            pl.BlockSpec((EPAD, HPAD), lambda i: (0, 0)),
            pl.BlockSpec((8, HPAD), lambda i: (0, 0)),
        ],
        out_specs=pl.BlockSpec((bb, 2), lambda i: (i, 0)),
        out_shape=jax.ShapeDtypeStruct((BATCH // 2, 2), jnp.float32),
    )(pooled2, w1p, aux)


def kernel(x, embed_table, W1, b1, bn_scale, bn_bias, bn_mean, bn_var, W2, b2):
    f32 = jnp.float32
    xt = x.astype(jnp.int32).T
    tpad = (
        jnp.zeros((VOCAB, EPAD), jnp.bfloat16)
        .at[:, :EMBED].set(embed_table.astype(jnp.bfloat16))
    )
    pooled2 = _sc_pool(tpad, xt)

    w1p = jnp.zeros((EPAD, HPAD), f32).at[:EMBED, :HIDDEN].set(W1)
    w1p = w1p[jnp.asarray(_PERM), :]

    pad1 = (0, HPAD - HIDDEN)
    aux = jnp.stack([
        jnp.pad(b1, pad1),
        jnp.pad(bn_scale, pad1),
        jnp.pad(bn_bias, pad1),
        jnp.pad(bn_mean, pad1),
        jnp.pad(bn_var, pad1, constant_values=1.0),
        jnp.pad(W2[:, 0], pad1),
        jnp.full((HPAD,), b2[0], dtype=f32),
        jnp.zeros((HPAD,), dtype=f32),
    ])

    out = _mlp(pooled2, w1p, aux)
    return out.reshape(BATCH)


# speedup vs baseline: 1.1707x; 1.0258x over previous
import functools

import jax
import jax.numpy as jnp
import numpy as np
from jax import lax
from jax.experimental import pallas as pl
from jax.experimental.pallas import tpu as pltpu
from jax.experimental.pallas import tpu_sc as plsc

VOCAB = 20000
EMBED = 50
HIDDEN = 100
BATCH = 4096
SEQLEN = 200
BN_EPS = 1e-5

NC = 2
NS = 16
LANES = 16
NW = NC * NS
BPW = BATCH // NW
NBUF = 4
EPAD = 64
HPAD = 128

CHUNK_STARTS = list(range(0, SEQLEN - LANES, LANES)) + [SEQLEN - LANES]
NCHUNK = len(CHUNK_STARTS)
BUF_ROWS = NCHUNK * LANES

_PERM = np.concatenate([
    np.arange(0, 32, 2), np.arange(1, 32, 2),
    np.arange(32, 64, 2), np.arange(33, 64, 2),
])


def _sc_pool(table, xt):
    mesh = plsc.VectorSubcoreMesh(core_axis_name="c", subcore_axis_name="s")

    @functools.partial(
        pl.kernel,
        out_type=jax.ShapeDtypeStruct((BATCH // 2, 2 * EPAD), jnp.float32),
        mesh=mesh,
        scratch_types=[
            pltpu.VMEM((SEQLEN, BPW), jnp.int32),
            *[pltpu.VMEM((BUF_ROWS,), jnp.int32) for _ in range(NBUF)],
            *[pltpu.VMEM((SEQLEN, EPAD), jnp.bfloat16) for _ in range(NBUF)],
            pltpu.VMEM((BPW // 2, 2 * EPAD), jnp.float32),
            *[pltpu.SemaphoreType.DMA for _ in range(NBUF)],
        ],
        compiler_params=pltpu.CompilerParams(
            needs_layout_passes=False, use_tc_tiling_on_sc=False
        ),
    )
    def pool(table_hbm, xt_hbm, out_hbm, raw_v, *rest):
        idxbufs = rest[:NBUF]
        bufs = rest[NBUF:2 * NBUF]
        acc_v = rest[2 * NBUF]
        sems = rest[2 * NBUF + 1:]
        wid = lax.axis_index("s") * NC + lax.axis_index("c")
        base = wid * BPW

        pltpu.sync_copy(xt_hbm.at[:, pl.ds(base, BPW)], raw_v)

        iot = lax.iota(jnp.int32, LANES)
        CH0 = 104

        def fire(e, b):
            ev = jnp.broadcast_to(e, (LANES,))
            for l0 in CHUNK_STARTS:
                idxbufs[b][pl.ds(l0, LANES)] = plsc.load_gather(
                    raw_v, [l0 + iot, ev]
                )
            pltpu.async_copy(
                table_hbm.at[idxbufs[b].at[pl.ds(0, CH0)]],
                bufs[b].at[pl.ds(0, CH0)],
                sems[b],
            )
            pltpu.async_copy(
                table_hbm.at[idxbufs[b].at[pl.ds(CH0, SEQLEN - CH0)]],
                bufs[b].at[pl.ds(CH0, SEQLEN - CH0)],
                sems[b],
            )

        for b in range(NBUF):
            fire(jnp.int32(b), b)

        inv = jnp.float32(1.0 / SEQLEN)
        zero = jnp.zeros((LANES,), jnp.float32)

        def reduce_block(buf):
            def grp(g, carry):
                a0, a1, a2, a3 = carry
                r0 = g * 8
                for r in range(8):
                    c0 = buf[r0 + r, pl.ds(0, 2 * LANES)]
                    c1 = buf[r0 + r, pl.ds(2 * LANES, 2 * LANES)]
                    e0, o0 = plsc.unpack(c0, format=plsc.PackFormat.INTERLEAVED)
                    e1, o1 = plsc.unpack(c1, format=plsc.PackFormat.INTERLEAVED)
                    a0 = a0 + e0
                    a1 = a1 + o0
                    a2 = a2 + e1
                    a3 = a3 + o1
                return a0, a1, a2, a3

            return lax.fori_loop(0, SEQLEN // 8, grp, (zero, zero, zero, zero))

        def outer(i, _):
            for b in range(NBUF):
                e = i * NBUF + b
                pltpu.make_async_copy(
                    table_hbm.at[pl.ds(0, SEQLEN)], bufs[b], sems[b]
                ).wait()
                a0, a1, a2, a3 = reduce_block(bufs[b])

                @pl.when(e + NBUF < BPW)
                def _():
                    fire(e + NBUF, b)

                row = i * (NBUF // 2) + (b // 2)
                col0 = (b % 2) * EPAD
                acc_v[row, pl.ds(col0, LANES)] = a0 * inv
                acc_v[row, pl.ds(col0 + LANES, LANES)] = a1 * inv
                acc_v[row, pl.ds(col0 + 2 * LANES, LANES)] = a2 * inv
                acc_v[row, pl.ds(col0 + 3 * LANES, LANES)] = a3 * inv
            return 0

        lax.fori_loop(0, BPW // NBUF, outer, 0)
        pltpu.sync_copy(acc_v, out_hbm.at[pl.ds(wid * (BPW // 2), BPW // 2)])

    return pool(table, xt)


def _mlp(pooled2, w1p, aux):
    def body(p_ref, w1_ref, aux_ref, o_ref):
        h = p_ref[...]
        h = jnp.where(h >= 0, h, 0.01 * h)
        b1 = aux_ref[0:1, :]
        s = aux_ref[1:2, :] * lax.rsqrt(aux_ref[4:5, :] + BN_EPS)
        t = aux_ref[2:3, :] - aux_ref[3:4, :] * s
        w2 = aux_ref[5:6, :]
        b2v = aux_ref[6:7, 0:1]

        def head(hh):
            h1 = jnp.dot(hh, w1_ref[...], preferred_element_type=jnp.float32) + b1
            h1 = jnp.where(h1 >= 0, h1, 0.01 * h1)
            h1 = h1 * s + t
            return jnp.sum(h1 * w2, axis=1, keepdims=True) + b2v

        o_ref[:, 0:1] = head(h[:, 0:EPAD])
        o_ref[:, 1:2] = head(h[:, EPAD:2 * EPAD])

    grid = 8
    bb = BATCH // 2 // grid
    return pl.pallas_call(
        body,
        grid=(grid,),
        in_specs=[
            pl.BlockSpec((bb, 2 * EPAD), lambda i: (i, 0)),
            pl.BlockSpec((EPAD, HPAD), lambda i: (0, 0)),
            pl.BlockSpec((8, HPAD), lambda i: (0, 0)),
        ],
        out_specs=pl.BlockSpec((bb, 2), lambda i: (i, 0)),
        out_shape=jax.ShapeDtypeStruct((BATCH // 2, 2), jnp.float32),
    )(pooled2, w1p, aux)


def kernel(x, embed_table, W1, b1, bn_scale, bn_bias, bn_mean, bn_var, W2, b2):
    f32 = jnp.float32
    xt = x.astype(jnp.int32).T
    tpad = (
        jnp.zeros((VOCAB, EPAD), jnp.bfloat16)
        .at[:, :EMBED].set(embed_table.astype(jnp.bfloat16))
    )
    pooled2 = _sc_pool(tpad, xt)

    w1p = jnp.zeros((EPAD, HPAD), f32).at[:EMBED, :HIDDEN].set(W1)
    w1p = w1p[jnp.asarray(_PERM), :]

    pad1 = (0, HPAD - HIDDEN)
    aux = jnp.stack([
        jnp.pad(b1, pad1),
        jnp.pad(bn_scale, pad1),
        jnp.pad(bn_bias, pad1),
        jnp.pad(bn_mean, pad1),
        jnp.pad(bn_var, pad1, constant_values=1.0),
        jnp.pad(W2[:, 0], pad1),
        jnp.full((HPAD,), b2[0], dtype=f32),
        jnp.zeros((HPAD,), dtype=f32),
    ])

    out = _mlp(pooled2, w1p, aux)
    return out.reshape(BATCH)


# speedup vs baseline: 1.3297x; 1.1359x over previous
import functools

import jax
import jax.numpy as jnp
import numpy as np
from jax import lax
from jax.experimental import pallas as pl
from jax.experimental.pallas import tpu as pltpu
from jax.experimental.pallas import tpu_sc as plsc

VOCAB = 20000
EMBED = 50
HIDDEN = 100
BATCH = 4096
SEQLEN = 200
BN_EPS = 1e-5

NC = 2
NS = 16
LANES = 16
NW = NC * NS
BPW = BATCH // NW
NBUF = 4
EPAD = 64
HPAD = 128

CHUNK_STARTS = list(range(0, SEQLEN - LANES, LANES)) + [SEQLEN - LANES]
NCHUNK = len(CHUNK_STARTS)
BUF_ROWS = NCHUNK * LANES

_PERM = np.concatenate([
    np.arange(0, 32, 2), np.arange(1, 32, 2),
    np.arange(32, 64, 2), np.arange(33, 64, 2),
])


def _sc_pool(table, xt):
    mesh = plsc.VectorSubcoreMesh(core_axis_name="c", subcore_axis_name="s")

    @functools.partial(
        pl.kernel,
        out_type=jax.ShapeDtypeStruct((BATCH // 2, 2 * EPAD), jnp.float32),
        mesh=mesh,
        scratch_types=[
            pltpu.VMEM((SEQLEN, BPW), jnp.int32),
            *[pltpu.VMEM((BUF_ROWS,), jnp.int32) for _ in range(NBUF)],
            *[pltpu.VMEM((SEQLEN, EPAD), jnp.bfloat16) for _ in range(NBUF)],
            pltpu.VMEM((BPW // 2, 2 * EPAD), jnp.float32),
            *[pltpu.SemaphoreType.DMA for _ in range(NBUF)],
        ],
        compiler_params=pltpu.CompilerParams(
            needs_layout_passes=False, use_tc_tiling_on_sc=False
        ),
    )
    def pool(table_hbm, xt_hbm, out_hbm, raw_v, *rest):
        idxbufs = rest[:NBUF]
        bufs = rest[NBUF:2 * NBUF]
        acc_v = rest[2 * NBUF]
        sems = rest[2 * NBUF + 1:]
        wid = lax.axis_index("s") * NC + lax.axis_index("c")
        base = wid * BPW

        pltpu.sync_copy(xt_hbm.at[:, pl.ds(base, BPW)], raw_v)

        iot = lax.iota(jnp.int32, LANES)
        CH0 = 104

        def fire(e, b):
            ev = jnp.broadcast_to(e, (LANES,))
            for l0 in CHUNK_STARTS:
                idxbufs[b][pl.ds(l0, LANES)] = plsc.load_gather(
                    raw_v, [l0 + iot, ev]
                )
            pltpu.async_copy(
                table_hbm.at[idxbufs[b].at[pl.ds(0, CH0)]],
                bufs[b].at[pl.ds(0, CH0)],
                sems[b],
            )
            pltpu.async_copy(
                table_hbm.at[idxbufs[b].at[pl.ds(CH0, SEQLEN - CH0)]],
                bufs[b].at[pl.ds(CH0, SEQLEN - CH0)],
                sems[b],
            )

        for b in range(NBUF):
            fire(jnp.int32(b), b)

        inv = jnp.float32(1.0 / SEQLEN)
        zero = jnp.zeros((LANES,), jnp.float32)

        def reduce_block(buf):
            def grp(g, carry):
                a0, a1, a2, a3 = carry
                r0 = g * 8
                for q in range(2):
                    r = r0 + 4 * q
                    s0 = (buf[r, pl.ds(0, 2 * LANES)]
                          + buf[r + 1, pl.ds(0, 2 * LANES)]) + (
                         buf[r + 2, pl.ds(0, 2 * LANES)]
                          + buf[r + 3, pl.ds(0, 2 * LANES)])
                    s1 = (buf[r, pl.ds(2 * LANES, 2 * LANES)]
                          + buf[r + 1, pl.ds(2 * LANES, 2 * LANES)]) + (
                         buf[r + 2, pl.ds(2 * LANES, 2 * LANES)]
                          + buf[r + 3, pl.ds(2 * LANES, 2 * LANES)])
                    e0, o0 = plsc.unpack(s0, format=plsc.PackFormat.INTERLEAVED)
                    e1, o1 = plsc.unpack(s1, format=plsc.PackFormat.INTERLEAVED)
                    a0 = a0 + e0
                    a1 = a1 + o0
                    a2 = a2 + e1
                    a3 = a3 + o1
                return a0, a1, a2, a3

            return lax.fori_loop(0, SEQLEN // 8, grp, (zero, zero, zero, zero))

        def outer(i, _):
            for b in range(NBUF):
                e = i * NBUF + b
                pltpu.make_async_copy(
                    table_hbm.at[pl.ds(0, SEQLEN)], bufs[b], sems[b]
                ).wait()
                a0, a1, a2, a3 = reduce_block(bufs[b])

                @pl.when(e + NBUF < BPW)
                def _():
                    fire(e + NBUF, b)

                row = i * (NBUF // 2) + (b // 2)
                col0 = (b % 2) * EPAD
                acc_v[row, pl.ds(col0, LANES)] = a0 * inv
                acc_v[row, pl.ds(col0 + LANES, LANES)] = a1 * inv
                acc_v[row, pl.ds(col0 + 2 * LANES, LANES)] = a2 * inv
                acc_v[row, pl.ds(col0 + 3 * LANES, LANES)] = a3 * inv
            return 0

        lax.fori_loop(0, BPW // NBUF, outer, 0)
        pltpu.sync_copy(acc_v, out_hbm.at[pl.ds(wid * (BPW // 2), BPW // 2)])

    return pool(table, xt)


def _mlp(pooled2, w1p, aux):
    def body(p_ref, w1_ref, aux_ref, o_ref):
        h = p_ref[...]
        h = jnp.where(h >= 0, h, 0.01 * h)
        b1 = aux_ref[0:1, :]
        s = aux_ref[1:2, :] * lax.rsqrt(aux_ref[4:5, :] + BN_EPS)
        t = aux_ref[2:3, :] - aux_ref[3:4, :] * s
        w2 = aux_ref[5:6, :]
        b2v = aux_ref[6:7, 0:1]

        def head(hh):
            h1 = jnp.dot(hh, w1_ref[...], preferred_element_type=jnp.float32) + b1
            h1 = jnp.where(h1 >= 0, h1, 0.01 * h1)
            h1 = h1 * s + t
            return jnp.sum(h1 * w2, axis=1, keepdims=True) + b2v

        o_ref[:, 0:1] = head(h[:, 0:EPAD])
        o_ref[:, 1:2] = head(h[:, EPAD:2 * EPAD])

    grid = 2
    bb = BATCH // 2 // grid
    return pl.pallas_call(
        body,
        grid=(grid,),
        in_specs=[
            pl.BlockSpec((bb, 2 * EPAD), lambda i: (i, 0)),
            pl.BlockSpec((EPAD, HPAD), lambda i: (0, 0)),
            pl.BlockSpec((8, HPAD), lambda i: (0, 0)),
        ],
        out_specs=pl.BlockSpec((bb, 2), lambda i: (i, 0)),
        out_shape=jax.ShapeDtypeStruct((BATCH // 2, 2), jnp.float32),
    )(pooled2, w1p, aux)


def kernel(x, embed_table, W1, b1, bn_scale, bn_bias, bn_mean, bn_var, W2, b2):
    f32 = jnp.float32
    xt = x.astype(jnp.int32).T
    tpad = (
        jnp.zeros((VOCAB, EPAD), jnp.bfloat16)
        .at[:, :EMBED].set(embed_table.astype(jnp.bfloat16))
    )
    pooled2 = _sc_pool(tpad, xt)

    w1p = jnp.zeros((EPAD, HPAD), f32).at[:EMBED, :HIDDEN].set(W1)
    w1p = w1p[jnp.asarray(_PERM), :]

    pad1 = (0, HPAD - HIDDEN)
    aux = jnp.stack([
        jnp.pad(b1, pad1),
        jnp.pad(bn_scale, pad1),
        jnp.pad(bn_bias, pad1),
        jnp.pad(bn_mean, pad1),
        jnp.pad(bn_var, pad1, constant_values=1.0),
        jnp.pad(W2[:, 0], pad1),
        jnp.full((HPAD,), b2[0], dtype=f32),
        jnp.zeros((HPAD,), dtype=f32),
    ])

    out = _mlp(pooled2, w1p, aux)
    return out.reshape(BATCH)
